# baseline (speedup 1.0000x reference)
import functools

import jax
import jax.numpy as jnp
from jax import lax
from jax.experimental import pallas as pl
from jax.experimental.pallas import tpu as pltpu
from jax.experimental.pallas import tpu_sc as plsc

N_ATOMS = 100000
N_TYPES = 50
CHUNK = 800
N_CHUNKS = N_ATOMS // CHUNK
GROUPS = CHUNK // 16
CHUNK_F = CHUNK * N_TYPES
NW = 32


@functools.partial(
    pl.kernel,
    out_type=jax.ShapeDtypeStruct((N_ATOMS * N_TYPES,), jnp.float32),
    mesh=plsc.VectorSubcoreMesh(core_axis_name="c", subcore_axis_name="s"),
    scratch_types=[
        pltpu.VMEM((CHUNK,), jnp.int32),
        pltpu.VMEM((CHUNK_F,), jnp.float32),
    ],
    compiler_params=pltpu.CompilerParams(needs_layout_passes=False),
)
def _onehot_sc(types_hbm, out_hbm, types_v, buf):
    wid = lax.axis_index("s") * 2 + lax.axis_index("c")
    ones16 = jnp.ones((16,), jnp.float32)
    zeros16 = jnp.zeros((16,), jnp.float32)
    iota16 = lax.iota(jnp.int32, 16)
    iota_row = iota16 * N_TYPES

    def zero_body(i, carry):
        for j in range(20):
            buf[pl.ds(i * 320 + j * 16, 16)] = zeros16
        return carry

    lax.fori_loop(0, CHUNK_F // 320, zero_body, 0)

    n_my = (N_CHUNKS - wid + NW - 1) // NW

    def chunk_body(i, carry):
        c = wid + i * NW
        base = c * CHUNK
        pltpu.sync_copy(types_hbm.at[pl.ds(base, CHUNK)], types_v)
        for g in range(GROUPS):
            t = types_v[pl.ds(g * 16, 16)]
            idx = t + (iota_row + g * CHUNK * N_TYPES // GROUPS)
            plsc.store_scatter(buf, [idx], ones16)
        pltpu.sync_copy(buf, out_hbm.at[pl.ds(base * N_TYPES, CHUNK_F)])
        for g in range(GROUPS):
            t = types_v[pl.ds(g * 16, 16)]
            idx = t + (iota_row + g * CHUNK * N_TYPES // GROUPS)
            plsc.store_scatter(buf, [idx], zeros16)
        return carry

    lax.fori_loop(0, n_my, chunk_body, 0)


def kernel(atom_types, pos):
    del pos
    types = atom_types.reshape(-1)
    out = _onehot_sc(types).reshape(N_ATOMS, N_TYPES)
    return (out, out)


# speedup vs baseline: 1.4427x; 1.4427x over previous
import functools

import jax
import jax.numpy as jnp
from jax import lax
from jax.experimental import pallas as pl
from jax.experimental.pallas import tpu as pltpu
from jax.experimental.pallas import tpu_sc as plsc

N_ATOMS = 100000
N_TYPES = 50
CHUNK = 800
N_CHUNKS = N_ATOMS // CHUNK
GROUPS = CHUNK // 16
NW = 32


@functools.partial(
    pl.kernel,
    out_type=jax.ShapeDtypeStruct((N_ATOMS, N_TYPES), jnp.float32),
    mesh=plsc.VectorSubcoreMesh(core_axis_name="c", subcore_axis_name="s"),
    scratch_types=[
        pltpu.VMEM((CHUNK,), jnp.int32),
        pltpu.VMEM((CHUNK, N_TYPES), jnp.float32),
    ],
    compiler_params=pltpu.CompilerParams(needs_layout_passes=False),
)
def _onehot_sc(types_hbm, zeros_hbm, out_hbm, types_v, buf):
    wid = lax.axis_index("s") * 2 + lax.axis_index("c")
    ones16 = jnp.ones((16,), jnp.float32)
    zeros16 = jnp.zeros((16,), jnp.float32)
    iota16 = lax.iota(jnp.int32, 16)

    pltpu.sync_copy(zeros_hbm, buf)

    n_my = (N_CHUNKS - wid + NW - 1) // NW

    def chunk_body(i, carry):
        c = wid + i * NW
        base = c * CHUNK
        pltpu.sync_copy(types_hbm.at[pl.ds(base, CHUNK)], types_v)
        for g in range(GROUPS):
            t = types_v[pl.ds(g * 16, 16)]
            plsc.store_scatter(buf, [iota16 + g * 16, t], ones16)
        pltpu.sync_copy(buf, out_hbm.at[pl.ds(base, CHUNK), :])
        for g in range(GROUPS):
            t = types_v[pl.ds(g * 16, 16)]
            plsc.store_scatter(buf, [iota16 + g * 16, t], zeros16)
        return carry

    lax.fori_loop(0, n_my, chunk_body, 0)


def kernel(atom_types, pos):
    del pos
    types = atom_types.reshape(-1)
    zeros = jnp.zeros((CHUNK, N_TYPES), jnp.float32)
    out = _onehot_sc(types, zeros)
    return (out, out)


# speedup vs baseline: 1.4445x; 1.0013x over previous
import functools

import jax
import jax.numpy as jnp
from jax import lax
from jax.experimental import pallas as pl
from jax.experimental.pallas import tpu as pltpu
from jax.experimental.pallas import tpu_sc as plsc

N_ATOMS = 100000
N_TYPES = 50
CHUNK = 800
N_CHUNKS = N_ATOMS // CHUNK
GROUPS = CHUNK // 16
NW = 32


@functools.partial(
    pl.kernel,
    out_type=jax.ShapeDtypeStruct((N_ATOMS, N_TYPES), jnp.float32),
    mesh=plsc.VectorSubcoreMesh(core_axis_name="c", subcore_axis_name="s"),
    scratch_types=[
        pltpu.VMEM((CHUNK,), jnp.int32),
        pltpu.VMEM((CHUNK, N_TYPES), jnp.float32),
    ],
    compiler_params=pltpu.CompilerParams(
        needs_layout_passes=False, use_tc_tiling_on_sc=True
    ),
)
def _onehot_sc(types_hbm, zeros_hbm, out_hbm, types_v, buf):
    wid = lax.axis_index("s") * 2 + lax.axis_index("c")
    ones16 = jnp.ones((16,), jnp.float32)
    zeros16 = jnp.zeros((16,), jnp.float32)
    iota16 = lax.iota(jnp.int32, 16)

    pltpu.sync_copy(zeros_hbm, buf)

    n_my = (N_CHUNKS - wid + NW - 1) // NW

    def chunk_body(i, carry):
        c = wid + i * NW
        base = c * CHUNK
        pltpu.sync_copy(types_hbm.at[pl.ds(base, CHUNK)], types_v)
        for g in range(GROUPS):
            t = types_v[pl.ds(g * 16, 16)]
            plsc.store_scatter(buf, [iota16 + g * 16, t], ones16)
        pltpu.sync_copy(buf, out_hbm.at[pl.ds(base, CHUNK), :])
        for g in range(GROUPS):
            t = types_v[pl.ds(g * 16, 16)]
            plsc.store_scatter(buf, [iota16 + g * 16, t], zeros16)
        return carry

    lax.fori_loop(0, n_my, chunk_body, 0)


def kernel(atom_types, pos):
    del pos
    types = atom_types.reshape(-1)
    zeros = jnp.zeros((CHUNK, N_TYPES), jnp.float32)
    out = _onehot_sc(types, zeros)
    return (out, out)


# speedup vs baseline: 2.0242x; 1.4013x over previous
import functools

import jax
import jax.numpy as jnp
from jax import lax
from jax.experimental import pallas as pl
from jax.experimental.pallas import tpu as pltpu
from jax.experimental.pallas import tpu_sc as plsc

N_ATOMS = 100000
N_TYPES = 50
CHUNK = 1024
N_FULL = N_ATOMS // CHUNK
N_BLOCKS = pl.cdiv(N_ATOMS, CHUNK)
GROUPS = CHUNK // 16
NW = 32


@functools.partial(
    pl.kernel,
    out_type=jax.ShapeDtypeStruct((N_TYPES, N_ATOMS), jnp.float32),
    mesh=plsc.VectorSubcoreMesh(core_axis_name="c", subcore_axis_name="s"),
    scratch_types=[
        pltpu.VMEM((CHUNK,), jnp.int32),
        pltpu.VMEM((N_TYPES, CHUNK), jnp.float32),
    ],
    compiler_params=pltpu.CompilerParams(needs_layout_passes=False),
)
def _onehot_sc(types_hbm, zeros_hbm, out_hbm, types_v, buf):
    wid = lax.axis_index("s") * 2 + lax.axis_index("c")
    ones16 = jnp.ones((16,), jnp.float32)
    zeros16 = jnp.zeros((16,), jnp.float32)
    iota16 = lax.iota(jnp.int32, 16)

    pltpu.sync_copy(zeros_hbm, buf)

    n_my = (N_FULL - wid + NW - 1) // NW

    def chunk_body(i, carry):
        c = wid + i * NW
        base = c * CHUNK
        pltpu.sync_copy(types_hbm.at[pl.ds(base, CHUNK)], types_v)
        for g in range(GROUPS):
            t = types_v[pl.ds(g * 16, 16)]
            plsc.store_scatter(buf, [t, iota16 + g * 16], ones16)
        pltpu.sync_copy(buf, out_hbm.at[:, pl.ds(base, CHUNK)])
        for g in range(GROUPS):
            t = types_v[pl.ds(g * 16, 16)]
            plsc.store_scatter(buf, [t, iota16 + g * 16], zeros16)
        return carry

    lax.fori_loop(0, n_my, chunk_body, 0)


def _onehot_block_tc(types_ref, o_ref):
    t = types_ref[:]
    rows = lax.broadcasted_iota(jnp.int32, (N_TYPES, CHUNK), 0)
    o_ref[...] = (rows == t[None, :]).astype(jnp.float32)


def _tail_tc(sc_ref, types_ref, o_ref):
    del sc_ref
    _onehot_block_tc(types_ref, o_ref)


_tail_call = pl.pallas_call(
    _tail_tc,
    grid=(1,),
    in_specs=[
        pl.BlockSpec(memory_space=pl.ANY),
        pl.BlockSpec((CHUNK,), lambda i: (N_BLOCKS - 1,)),
    ],
    out_specs=pl.BlockSpec((N_TYPES, CHUNK), lambda i: (0, N_BLOCKS - 1)),
    out_shape=jax.ShapeDtypeStruct((N_TYPES, N_ATOMS), jnp.float32),
    input_output_aliases={0: 0},
)

_dup_call = pl.pallas_call(
    _onehot_block_tc,
    grid=(N_BLOCKS,),
    in_specs=[pl.BlockSpec((CHUNK,), lambda i: (i,))],
    out_specs=pl.BlockSpec((N_TYPES, CHUNK), lambda i: (0, i)),
    out_shape=jax.ShapeDtypeStruct((N_TYPES, N_ATOMS), jnp.float32),
)


def kernel(atom_types, pos):
    del pos
    types = atom_types.reshape(-1)
    zeros = jnp.zeros((N_TYPES, CHUNK), jnp.float32)
    sc_out = _onehot_sc(types, zeros)
    out1 = _tail_call(sc_out, types)
    out2 = _dup_call(types)
    return (out1.T, out2.T)


# speedup vs baseline: 2.8077x; 1.3871x over previous
import functools

import jax
import jax.numpy as jnp
from jax import lax
from jax.experimental import pallas as pl
from jax.experimental.pallas import tpu as pltpu
from jax.experimental.pallas import tpu_sc as plsc

N_ATOMS = 100000
N_TYPES = 50
CHUNK = 1024
N_FULL = N_ATOMS // CHUNK
N_BLOCKS = pl.cdiv(N_ATOMS, CHUNK)
GROUPS = CHUNK // 16
NW = 32


@functools.partial(
    pl.kernel,
    out_type=jax.ShapeDtypeStruct((N_TYPES, N_ATOMS), jnp.float32),
    mesh=plsc.VectorSubcoreMesh(core_axis_name="c", subcore_axis_name="s"),
    scratch_types=[
        pltpu.VMEM((CHUNK,), jnp.int32),
        pltpu.VMEM((N_TYPES, CHUNK), jnp.float32),
    ],
    compiler_params=pltpu.CompilerParams(needs_layout_passes=False),
)
def _onehot_sc(types_hbm, zeros_hbm, out_hbm, types_v, buf):
    wid = lax.axis_index("s") * 2 + lax.axis_index("c")
    ones16 = jnp.ones((16,), jnp.float32)
    zeros16 = jnp.zeros((16,), jnp.float32)
    iota16 = lax.iota(jnp.int32, 16)

    pltpu.sync_copy(zeros_hbm, buf)

    n_my = (N_FULL - wid + NW - 1) // NW

    def chunk_body(i, carry):
        c = wid + i * NW
        base = c * CHUNK
        pltpu.sync_copy(types_hbm.at[pl.ds(base, CHUNK)], types_v)
        for g in range(GROUPS):
            t = types_v[pl.ds(g * 16, 16)]
            plsc.store_scatter(buf, [t, iota16 + g * 16], ones16)
        pltpu.sync_copy(buf, out_hbm.at[:, pl.ds(base, CHUNK)])
        for g in range(GROUPS):
            t = types_v[pl.ds(g * 16, 16)]
            plsc.store_scatter(buf, [t, iota16 + g * 16], zeros16)
        return carry

    lax.fori_loop(0, n_my, chunk_body, 0)


DUP_CHUNK = 8192
N_DUP_BLOCKS = pl.cdiv(N_ATOMS, DUP_CHUNK)


def _onehot_block_tc(types_ref, o_ref):
    t = types_ref[:]
    rows = lax.broadcasted_iota(jnp.int32, (N_TYPES, t.shape[0]), 0)
    o_ref[...] = (rows == t[None, :]).astype(jnp.float32)


def _tail_tc(sc_ref, types_ref, o_ref):
    del sc_ref
    _onehot_block_tc(types_ref, o_ref)


_tail_call = pl.pallas_call(
    _tail_tc,
    grid=(1,),
    in_specs=[
        pl.BlockSpec(memory_space=pl.ANY),
        pl.BlockSpec((CHUNK,), lambda i: (N_BLOCKS - 1,)),
    ],
    out_specs=pl.BlockSpec((N_TYPES, CHUNK), lambda i: (0, N_BLOCKS - 1)),
    out_shape=jax.ShapeDtypeStruct((N_TYPES, N_ATOMS), jnp.float32),
    input_output_aliases={0: 0},
)

_dup_call = pl.pallas_call(
    _onehot_block_tc,
    grid=(N_DUP_BLOCKS,),
    in_specs=[pl.BlockSpec((DUP_CHUNK,), lambda i: (i,))],
    out_specs=pl.BlockSpec((N_TYPES, DUP_CHUNK), lambda i: (0, i)),
    out_shape=jax.ShapeDtypeStruct((N_TYPES, N_ATOMS), jnp.float32),
)


def kernel(atom_types, pos):
    del pos
    types = atom_types.reshape(-1)
    zeros = jnp.zeros((N_TYPES, CHUNK), jnp.float32)
    sc_out = _onehot_sc(types, zeros)
    out1 = _tail_call(sc_out, types)
    out2 = _dup_call(types)
    return (out1.T, out2.T)
